# baseline (speedup 1.0000x reference)
import functools

import jax
import jax.numpy as jnp
from jax import lax
from jax.experimental import pallas as pl
from jax.experimental.pallas import tpu as pltpu
from jax.experimental.pallas import tpu_sc as plsc

N_ENT = 10000
N_PAD = 10240
N_USERS = 4096
N_REL = 11
D = 128
E = 320000
NW = 32
EPT = E // NW
CH = 80
NCH = EPT // CH
F32 = jnp.float32
I32 = jnp.int32
_NEG = -3e38


def _a16_body(e2_ref, w2_ref, o_ref):
    o_ref[...] = lax.dot_general(
        e2_ref[...], w2_ref[...], (((1,), (1,)), ((), ())),
        precision=lax.Precision.HIGHEST)


def _combine_body(m_ref, s_ref, om_ref, oi_ref):
    mp = m_ref[...]
    sp = s_ref[...]
    m = jnp.max(mp, axis=0, keepdims=True)
    s = jnp.sum(sp * jnp.exp(mp - m), axis=0, keepdims=True)
    om_ref[...] = m
    oi_ref[...] = 1.0 / s


def _final_body(p0_ref, p1_ref, o_ref):
    o_ref[...] = p0_ref[...] + p1_ref[...]


def _scale_table_body(e_ref, w_ref, o_ref):
    r = pl.program_id(0)
    o_ref[...] = e_ref[...] * w_ref[pl.ds(r, 1), :]


def _user_body(ia_ref, ee_ref, ue_ref, wp_ref, o_ref):
    ua = lax.dot_general(ia_ref[...], ee_ref[...], (((1,), (0,)), ((), ())),
                         precision=lax.Precision.HIGHEST)
    logits = lax.dot_general(ue_ref[...], wp_ref[...], (((1,), (1,)), ((), ())),
                             precision=lax.Precision.HIGHEST)
    col = lax.broadcasted_iota(I32, logits.shape, 1)
    logits = jnp.where(col < N_REL, logits, -1e30)
    logits = logits - jnp.max(logits, axis=-1, keepdims=True)
    p = jnp.exp(logits)
    p = p / jnp.sum(p, axis=-1, keepdims=True)
    sw = lax.dot_general(p, wp_ref[...], (((1,), (0,)), ((), ())),
                         precision=lax.Precision.HIGHEST)
    o_ref[...] = ua + sw * ua


def _perm16(x, idx):
    return jnp.take_along_axis(x, idx, axis=0, mode="promise_in_bounds")


def _seg_scatter(tab, h, v, is_max):
    iota = lax.iota(I32, 16)
    hs, perm = plsc.sort_key_val(h, iota)
    x = _perm16(v, perm)
    for d in (1, 2, 4, 8):
        src = jnp.maximum(iota - d, 0)
        same = (_perm16(hs, src) == hs) & (iota >= d)
        xsh = _perm16(x, src)
        comb = jnp.maximum(x, xsh) if is_max else x + xsh
        x = jnp.where(same, comb, x)
    is_last = (iota == 15) | (_perm16(hs, jnp.minimum(iota + 1, 15)) != hs)
    cur = plsc.load_gather(tab, [hs], mask=is_last)
    new = jnp.maximum(cur, x) if is_max else cur + x
    plsc.store_scatter(tab, [hs], new, mask=is_last)


_MESH = plsc.VectorSubcoreMesh(core_axis_name="c", subcore_axis_name="s")


@functools.partial(
    pl.kernel,
    out_type=(
        jax.ShapeDtypeStruct((NW, N_PAD), F32),
        jax.ShapeDtypeStruct((NW, N_PAD), F32),
        jax.ShapeDtypeStruct((NW, EPT), F32),
    ),
    mesh=_MESH,
    scratch_types=(
        pltpu.VMEM((EPT,), I32),
        pltpu.VMEM((EPT,), I32),
        pltpu.VMEM((EPT,), F32),
        pltpu.VMEM((N_PAD,), F32),
        pltpu.VMEM((N_PAD,), F32),
        pltpu.VMEM((CH,), F32),
        pltpu.VMEM((CH,), F32),
        pltpu.VMEM((CH,), F32),
        pltpu.VMEM((CH,), F32),
        pltpu.SemaphoreType.DMA,
        pltpu.SemaphoreType.DMA,
    ),
    compiler_params=pltpu.CompilerParams(needs_layout_passes=False),
)
def _sc_stats(a16f, hidxr, tidxr, m_out, s_out, att_out,
              hidx_v, tidx_v, att_v, m_tab, s_tab, hv0, tv0, hv1, tv1,
              sem0, sem1):
    cid = lax.axis_index("c")
    sid = lax.axis_index("s")
    wid = sid * 2 + cid
    pltpu.sync_copy(hidxr.at[wid], hidx_v)
    pltpu.sync_copy(tidxr.at[wid], tidx_v)

    def _init(i, _):
        m_tab[pl.ds(i * 16, 16)] = jnp.full((16,), _NEG, F32)
        s_tab[pl.ds(i * 16, 16)] = jnp.zeros((16,), F32)
        return 0
    lax.fori_loop(0, N_PAD // 16, _init, 0)

    def _issue(ci, hv, tv, sem):
        pltpu.async_copy(a16f.at[hidx_v.at[pl.ds(ci * CH, CH)]], hv, sem)
        pltpu.async_copy(a16f.at[tidx_v.at[pl.ds(ci * CH, CH)]], tv, sem)

    def _wait(hv, tv, sem):
        pltpu.make_async_copy(a16f.at[pl.ds(0, CH)], hv, sem).wait()
        pltpu.make_async_copy(a16f.at[pl.ds(0, CH)], tv, sem).wait()

    def _proc(ci, hv, tv):
        base = ci * CH

        def _vec(vi, _):
            off = vi * 16
            av = hv[pl.ds(off, 16)] * tv[pl.ds(off, 16)]
            att_v[pl.ds(base + off, 16)] = av
            h = lax.shift_right_logical(hidx_v[pl.ds(base + off, 16)], 4)
            _seg_scatter(m_tab, h, av, True)
            return 0
        lax.fori_loop(0, CH // 16, _vec, 0)

    _issue(0, hv0, tv0, sem0)

    def _pair(q, _):
        ci0 = q * 2
        _issue(ci0 + 1, hv1, tv1, sem1)
        _wait(hv0, tv0, sem0)
        _proc(ci0, hv0, tv0)
        _issue(ci0 + 2, hv0, tv0, sem0)
        _wait(hv1, tv1, sem1)
        _proc(ci0 + 1, hv1, tv1)
        return 0
    lax.fori_loop(0, (NCH - 1) // 2, _pair, 0)
    _wait(hv0, tv0, sem0)
    _proc(NCH - 1, hv0, tv0)

    def _chunk2(ei, _):
        off = ei * 16
        h = lax.shift_right_logical(hidx_v[pl.ds(off, 16)], 4)
        av = att_v[pl.ds(off, 16)]
        mh = plsc.load_gather(m_tab, [h])
        ev = jnp.exp(av - mh)
        _seg_scatter(s_tab, h, ev, False)
        return 0
    lax.fori_loop(0, EPT // 16, _chunk2, 0)

    pltpu.sync_copy(m_tab, m_out.at[wid])
    pltpu.sync_copy(s_tab, s_out.at[wid])
    pltpu.sync_copy(att_v, att_out.at[wid])


N_PASS = 4
NR_RANGE = N_PAD // N_PASS
CH2 = 128
NCH2 = (EPT + CH2 - 1) // CH2
LPAD = NCH2 * CH2


@functools.partial(
    pl.kernel,
    out_type=jax.ShapeDtypeStruct((2, N_PAD, D), F32),
    mesh=_MESH,
    scratch_types=(
        pltpu.VMEM((EPT,), I32),
        pltpu.VMEM((EPT,), I32),
        pltpu.VMEM((EPT,), F32),
        pltpu.VMEM((N_PAD,), F32),
        pltpu.VMEM((N_PAD,), F32),
        pltpu.VMEM((LPAD,), I32),
        pltpu.VMEM((LPAD,), F32),
        pltpu.VMEM((LPAD,), I32),
        pltpu.VMEM((NCH2, CH2), I32),
        pltpu.VMEM((CH2, D), F32),
        pltpu.VMEM_SHARED((NR_RANGE, D), F32),
    ),
    compiler_params=pltpu.CompilerParams(needs_layout_passes=False),
)
def _sc_agg(tab, headr, trowr, attr, mr, invr, out,
            head_f, trow_f, att_f, m_v, inv_v,
            cl_tr, cl_wv, cl_hd, cl_hd2, rows0, accum):
    cid = lax.axis_index("c")
    sid = lax.axis_index("s")
    wid = sid * 2 + cid
    pltpu.sync_copy(headr.at[wid], head_f)
    pltpu.sync_copy(trowr.at[wid], trow_f)
    pltpu.sync_copy(attr.at[wid], att_f)
    pltpu.sync_copy(mr, m_v)
    pltpu.sync_copy(invr, inv_v)
    rows_per_tile = NR_RANGE // 16

    for p in range(N_PASS):
        lo = p * NR_RANGE

        def _z(j, _):
            for k in range(8):
                rows0[j, pl.ds(k * 16, 16)] = jnp.zeros((16,), F32)
            return 0
        lax.fori_loop(0, 80, _z, 0)
        for b in range(rows_per_tile // 80):
            pltpu.sync_copy(
                rows0.at[pl.ds(0, 80)],
                accum.at[pl.ds(sid * rows_per_tile + b * 80, 80)])

        def _pf(i, _):
            sl = pl.ds(i * 16, 16)
            cl_tr[sl] = jnp.zeros((16,), I32)
            cl_wv[sl] = jnp.zeros((16,), F32)
            cl_hd[sl] = jnp.zeros((16,), I32)
            return 0
        lax.fori_loop(0, LPAD // 16, _pf, 0)

        def _cmp(vi, off):
            sl = pl.ds(vi * 16, 16)
            h = head_f[sl]
            av = att_f[sl]
            mh = plsc.load_gather(m_v, [h])
            ih = plsc.load_gather(inv_v, [h])
            wv = jnp.exp(av - mh) * ih
            msk = (h >= lo) & (h < lo + NR_RANGE)
            dst = pl.ds(off, 16)
            plsc.store_compressed(cl_tr.at[dst], trow_f[sl], mask=msk)
            plsc.store_compressed(cl_wv.at[dst], wv, mask=msk)
            plsc.store_compressed(cl_hd.at[dst], h - lo, mask=msk)
            cnt = plsc.all_reduce_population_count(msk)
            return off + cnt[0]
        n_p = lax.fori_loop(0, EPT // 16, _cmp, jnp.int32(0))

        def _cp(ci, _):
            for v in range(CH2 // 16):
                cl_hd2[ci, pl.ds(v * 16, 16)] = \
                    cl_hd[pl.ds(ci * CH2 + v * 16, 16)]
            return 0
        lax.fori_loop(0, NCH2, _cp, 0)

        plsc.subcore_barrier()

        def _chunkc(ci, _):
            @pl.when(ci * CH2 < n_p)
            def _do():
                base = ci * CH2
                pltpu.sync_copy(tab.at[cl_tr.at[pl.ds(base, CH2)]], rows0)

                @plsc.parallel_loop(0, CH2, 1, unroll=2)
                def _edge(j):
                    jb = jnp.full((16,), j, I32) + base
                    wj = plsc.load_gather(cl_wv, [jb])
                    for k in range(8):
                        v = rows0[j, pl.ds(k * 16, 16)]
                        rows0[j, pl.ds(k * 16, 16)] = v * wj
                pltpu.sync_copy(rows0, accum.at[cl_hd2.at[ci]], add=True)
            return 0
        lax.fori_loop(0, NCH2, _chunkc, 0)

        plsc.subcore_barrier()
        for b in range(rows_per_tile // 80):
            r0 = sid * rows_per_tile + b * 80
            pltpu.sync_copy(accum.at[pl.ds(r0, 80)],
                            out.at[cid, pl.ds(lo + r0, 80)])
        plsc.subcore_barrier()


def kernel(entity_emb, user_emb, edge_index, edge_type, interact_mat, weight):
    head = edge_index[0]
    tail = edge_index[1]
    rel = jnp.where(edge_type == 0, N_REL - 1, edge_type - 1).astype(I32)
    hidx = (head * 16 + rel).reshape(NW, EPT)
    tidx = (tail * 16 + rel).reshape(NW, EPT)
    e2p = jnp.pad(entity_emb * entity_emb, ((0, N_PAD - N_ENT), (0, 0)))
    w2p = jnp.pad(weight * weight, ((0, 16 - N_REL), (0, 0)))
    wp = jnp.pad(weight, ((0, 16 - N_REL), (0, 0)))
    trow = (rel * N_ENT + tail).reshape(NW, EPT)

    tab = pl.pallas_call(
        _scale_table_body,
        grid=(N_REL, 5),
        in_specs=[pl.BlockSpec((2000, D), lambda r, j: (j, 0)),
                  pl.BlockSpec((N_REL, D), lambda r, j: (0, 0))],
        out_specs=pl.BlockSpec((2000, D), lambda r, j: (r * 5 + j, 0)),
        out_shape=jax.ShapeDtypeStruct((N_REL * N_ENT, D), F32),
    )(entity_emb, weight)

    a16 = pl.pallas_call(
        _a16_body,
        grid=(8,),
        in_specs=[pl.BlockSpec((1280, D), lambda i: (i, 0)),
                  pl.BlockSpec((16, D), lambda i: (0, 0))],
        out_specs=pl.BlockSpec((1280, 16), lambda i: (i, 0)),
        out_shape=jax.ShapeDtypeStruct((N_PAD, 16), F32),
    )(e2p, w2p)

    m_part, s_part, att = _sc_stats(a16.reshape(N_PAD * 16), hidx, tidx)

    m2, inv2 = pl.pallas_call(
        _combine_body,
        grid=(8,),
        in_specs=[pl.BlockSpec((NW, 1280), lambda i: (0, i)),
                  pl.BlockSpec((NW, 1280), lambda i: (0, i))],
        out_specs=[pl.BlockSpec((1, 1280), lambda i: (0, i)),
                   pl.BlockSpec((1, 1280), lambda i: (0, i))],
        out_shape=[jax.ShapeDtypeStruct((1, N_PAD), F32),
                   jax.ShapeDtypeStruct((1, N_PAD), F32)],
    )(m_part, s_part)

    parts = _sc_agg(tab, head.reshape(NW, EPT), trow, att,
                    m2.reshape(N_PAD), inv2.reshape(N_PAD))

    entity_agg = pl.pallas_call(
        _final_body,
        grid=(8,),
        in_specs=[pl.BlockSpec((1280, D), lambda i: (i, 0)),
                  pl.BlockSpec((1280, D), lambda i: (i, 0))],
        out_specs=pl.BlockSpec((1280, D), lambda i: (i, 0)),
        out_shape=jax.ShapeDtypeStruct((N_PAD, D), F32),
    )(parts[0], parts[1])[:N_ENT]

    user_agg = pl.pallas_call(
        _user_body,
        grid=(16,),
        in_specs=[pl.BlockSpec((256, N_ENT), lambda i: (i, 0)),
                  pl.BlockSpec((N_ENT, D), lambda i: (0, 0)),
                  pl.BlockSpec((256, D), lambda i: (i, 0)),
                  pl.BlockSpec((16, D), lambda i: (0, 0))],
        out_specs=pl.BlockSpec((256, D), lambda i: (i, 0)),
        out_shape=jax.ShapeDtypeStruct((N_USERS, D), F32),
    )(interact_mat, entity_emb, user_emb, wp)

    return entity_agg, user_agg


# speedup vs baseline: 1.1493x; 1.1493x over previous
import functools

import jax
import jax.numpy as jnp
from jax import lax
from jax.experimental import pallas as pl
from jax.experimental.pallas import tpu as pltpu
from jax.experimental.pallas import tpu_sc as plsc

N_ENT = 10000
N_PAD = 10240
N_USERS = 4096
N_REL = 11
D = 128
E = 320000
NW = 32
EPT = E // NW
CH = 80
NCH = EPT // CH
F32 = jnp.float32
I32 = jnp.int32
_NEG = -3e38


def _a16_body(e2_ref, w2_ref, o_ref):
    o_ref[...] = lax.dot_general(
        e2_ref[...], w2_ref[...], (((1,), (1,)), ((), ())),
        precision=lax.Precision.HIGHEST)


def _combine_body(m_ref, s_ref, om_ref, oi_ref):
    mp = m_ref[...]
    sp = s_ref[...]
    m = jnp.max(mp, axis=0, keepdims=True)
    s = jnp.sum(sp * jnp.exp(mp - m), axis=0, keepdims=True)
    om_ref[...] = m
    oi_ref[...] = 1.0 / s


def _final_body(p0_ref, p1_ref, o_ref):
    o_ref[...] = p0_ref[...] + p1_ref[...]


def _scale_table_body(e_ref, w_ref, o_ref):
    r = pl.program_id(0)
    o_ref[...] = e_ref[...] * w_ref[pl.ds(r, 1), :]


def _user_body(ia_ref, ee_ref, ue_ref, wp_ref, o_ref):
    ua = lax.dot_general(ia_ref[...], ee_ref[...], (((1,), (0,)), ((), ())),
                         precision=lax.Precision.HIGHEST)
    logits = lax.dot_general(ue_ref[...], wp_ref[...], (((1,), (1,)), ((), ())),
                             precision=lax.Precision.HIGHEST)
    col = lax.broadcasted_iota(I32, logits.shape, 1)
    logits = jnp.where(col < N_REL, logits, -1e30)
    logits = logits - jnp.max(logits, axis=-1, keepdims=True)
    p = jnp.exp(logits)
    p = p / jnp.sum(p, axis=-1, keepdims=True)
    sw = lax.dot_general(p, wp_ref[...], (((1,), (0,)), ((), ())),
                         precision=lax.Precision.HIGHEST)
    o_ref[...] = ua + sw * ua


def _perm16(x, idx):
    return jnp.take_along_axis(x, idx, axis=0, mode="promise_in_bounds")


def _seg_scatter(tab, h, v, is_max):
    iota = lax.iota(I32, 16)
    hs, perm = plsc.sort_key_val(h, iota)
    x = _perm16(v, perm)
    for d in (1, 2, 4, 8):
        src = jnp.maximum(iota - d, 0)
        same = (_perm16(hs, src) == hs) & (iota >= d)
        xsh = _perm16(x, src)
        comb = jnp.maximum(x, xsh) if is_max else x + xsh
        x = jnp.where(same, comb, x)
    is_last = (iota == 15) | (_perm16(hs, jnp.minimum(iota + 1, 15)) != hs)
    cur = plsc.load_gather(tab, [hs], mask=is_last)
    new = jnp.maximum(cur, x) if is_max else cur + x
    plsc.store_scatter(tab, [hs], new, mask=is_last)


_MESH = plsc.VectorSubcoreMesh(core_axis_name="c", subcore_axis_name="s")


@functools.partial(
    pl.kernel,
    out_type=(
        jax.ShapeDtypeStruct((NW, N_PAD), F32),
        jax.ShapeDtypeStruct((NW, N_PAD), F32),
        jax.ShapeDtypeStruct((NW, EPT), F32),
    ),
    mesh=_MESH,
    scratch_types=(
        pltpu.VMEM((EPT,), I32),
        pltpu.VMEM((EPT,), I32),
        pltpu.VMEM((EPT,), F32),
        pltpu.VMEM((N_PAD,), F32),
        pltpu.VMEM((N_PAD,), F32),
        pltpu.VMEM((CH,), F32),
        pltpu.VMEM((CH,), F32),
        pltpu.VMEM((CH,), F32),
        pltpu.VMEM((CH,), F32),
        pltpu.SemaphoreType.DMA,
        pltpu.SemaphoreType.DMA,
    ),
    compiler_params=pltpu.CompilerParams(needs_layout_passes=False),
)
def _sc_stats(a16f, hidxr, tidxr, m_out, s_out, att_out,
              hidx_v, tidx_v, att_v, m_tab, s_tab, hv0, tv0, hv1, tv1,
              sem0, sem1):
    cid = lax.axis_index("c")
    sid = lax.axis_index("s")
    wid = sid * 2 + cid
    pltpu.sync_copy(hidxr.at[wid], hidx_v)
    pltpu.sync_copy(tidxr.at[wid], tidx_v)

    def _init(i, _):
        m_tab[pl.ds(i * 16, 16)] = jnp.full((16,), _NEG, F32)
        s_tab[pl.ds(i * 16, 16)] = jnp.zeros((16,), F32)
        return 0
    lax.fori_loop(0, N_PAD // 16, _init, 0)

    def _issue(ci, hv, tv, sem):
        pltpu.async_copy(a16f.at[hidx_v.at[pl.ds(ci * CH, CH)]], hv, sem)
        pltpu.async_copy(a16f.at[tidx_v.at[pl.ds(ci * CH, CH)]], tv, sem)

    def _wait(hv, tv, sem):
        pltpu.make_async_copy(a16f.at[pl.ds(0, CH)], hv, sem).wait()
        pltpu.make_async_copy(a16f.at[pl.ds(0, CH)], tv, sem).wait()

    def _proc(ci, hv, tv):
        base = ci * CH

        def _vec(vi, _):
            off = vi * 16
            av = hv[pl.ds(off, 16)] * tv[pl.ds(off, 16)]
            att_v[pl.ds(base + off, 16)] = av
            h = lax.shift_right_logical(hidx_v[pl.ds(base + off, 16)], 4)
            _seg_scatter(m_tab, h, av, True)
            return 0
        lax.fori_loop(0, CH // 16, _vec, 0)

    _issue(0, hv0, tv0, sem0)

    def _pair(q, _):
        ci0 = q * 2
        _issue(ci0 + 1, hv1, tv1, sem1)
        _wait(hv0, tv0, sem0)
        _proc(ci0, hv0, tv0)
        _issue(ci0 + 2, hv0, tv0, sem0)
        _wait(hv1, tv1, sem1)
        _proc(ci0 + 1, hv1, tv1)
        return 0
    lax.fori_loop(0, (NCH - 1) // 2, _pair, 0)
    _wait(hv0, tv0, sem0)
    _proc(NCH - 1, hv0, tv0)

    def _chunk2(ei, _):
        off = ei * 16
        h = lax.shift_right_logical(hidx_v[pl.ds(off, 16)], 4)
        av = att_v[pl.ds(off, 16)]
        mh = plsc.load_gather(m_tab, [h])
        ev = jnp.exp(av - mh)
        _seg_scatter(s_tab, h, ev, False)
        return 0
    lax.fori_loop(0, EPT // 16, _chunk2, 0)

    pltpu.sync_copy(m_tab, m_out.at[wid])
    pltpu.sync_copy(s_tab, s_out.at[wid])
    pltpu.sync_copy(att_v, att_out.at[wid])


NR_RANGE = N_PAD // 2


@functools.partial(
    pl.kernel,
    out_type=jax.ShapeDtypeStruct((2, N_PAD, D), F32),
    mesh=_MESH,
    scratch_types=(
        pltpu.VMEM((EPT,), I32),
        pltpu.VMEM((NCH, CH), I32),
        pltpu.VMEM((EPT,), I32),
        pltpu.VMEM((EPT,), F32),
        pltpu.VMEM((N_PAD,), F32),
        pltpu.VMEM((N_PAD,), F32),
        pltpu.VMEM((CH, D), F32),
        pltpu.VMEM((CH,), F32),
        pltpu.VMEM_SHARED((NR_RANGE, D), F32),
        pltpu.SemaphoreType.DMA,
        pltpu.SemaphoreType.DMA,
    ),
    compiler_params=pltpu.CompilerParams(needs_layout_passes=False),
)
def _sc_agg(tab, headr, headcr, trowr, attr, mr, invr, out,
            head_f, head_v, trow_f, att_f, m_v, inv_v,
            rows0, wbuf, accum, g0, g1):
    cid = lax.axis_index("c")
    sid = lax.axis_index("s")
    wid = sid * 2 + cid
    pltpu.sync_copy(headr.at[wid], head_f)
    pltpu.sync_copy(trowr.at[wid], trow_f)
    pltpu.sync_copy(attr.at[wid], att_f)
    pltpu.sync_copy(mr, m_v)
    pltpu.sync_copy(invr, inv_v)
    rows_per_tile = NR_RANGE // 16

    for p in range(2):
        lo = p * NR_RANGE
        pltpu.sync_copy(headcr.at[p, wid], head_v)

        def _z(j, _):
            for k in range(8):
                rows0[j, pl.ds(k * 16, 16)] = jnp.zeros((16,), F32)
            return 0
        lax.fori_loop(0, CH, _z, 0)
        for b in range(rows_per_tile // CH):
            pltpu.sync_copy(
                rows0, accum.at[pl.ds(sid * rows_per_tile + b * CH, CH)])
        plsc.subcore_barrier()

        def _proc(ci, rbuf):
            base = ci * CH

            @plsc.parallel_loop(0, CH // 16, 1, unroll=2)
            def _vec(vi):
                off = base + vi * 16
                h = head_f[pl.ds(off, 16)]
                av = att_f[pl.ds(off, 16)]
                mh = plsc.load_gather(m_v, [h])
                ih = plsc.load_gather(inv_v, [h])
                wv = jnp.exp(av - mh) * ih
                in_range = (h >= lo) & (h < lo + NR_RANGE)
                wbuf[pl.ds(vi * 16, 16)] = jnp.where(in_range, wv, 0.0)

            @plsc.parallel_loop(0, CH, 1, unroll=4)
            def _edge(j):
                jb = jnp.full((16,), j, I32)
                wj = plsc.load_gather(wbuf, [jb])
                for k in range(8):
                    v = rbuf[j, pl.ds(k * 16, 16)]
                    rbuf[j, pl.ds(k * 16, 16)] = v * wj
            pltpu.sync_copy(rbuf, accum.at[head_v.at[ci]], add=True)

        def _chunkc(ci, _):
            pltpu.sync_copy(tab.at[trow_f.at[pl.ds(ci * CH, CH)]], rows0)
            _proc(ci, rows0)
            return 0
        lax.fori_loop(0, NCH, _chunkc, 0)

        plsc.subcore_barrier()
        for b in range(rows_per_tile // CH):
            r0 = sid * rows_per_tile + b * CH
            pltpu.sync_copy(accum.at[pl.ds(r0, CH)],
                            out.at[cid, pl.ds(lo + r0, CH)])
        plsc.subcore_barrier()


def kernel(entity_emb, user_emb, edge_index, edge_type, interact_mat, weight):
    head = edge_index[0]
    tail = edge_index[1]
    rel = jnp.where(edge_type == 0, N_REL - 1, edge_type - 1).astype(I32)
    hidx = (head * 16 + rel).reshape(NW, EPT)
    tidx = (tail * 16 + rel).reshape(NW, EPT)
    e2p = jnp.pad(entity_emb * entity_emb, ((0, N_PAD - N_ENT), (0, 0)))
    w2p = jnp.pad(weight * weight, ((0, 16 - N_REL), (0, 0)))
    wp = jnp.pad(weight, ((0, 16 - N_REL), (0, 0)))
    trow = (rel * N_ENT + tail).reshape(NW, EPT)

    tab = pl.pallas_call(
        _scale_table_body,
        grid=(N_REL, 5),
        in_specs=[pl.BlockSpec((2000, D), lambda r, j: (j, 0)),
                  pl.BlockSpec((N_REL, D), lambda r, j: (0, 0))],
        out_specs=pl.BlockSpec((2000, D), lambda r, j: (r * 5 + j, 0)),
        out_shape=jax.ShapeDtypeStruct((N_REL * N_ENT, D), F32),
    )(entity_emb, weight)

    a16 = pl.pallas_call(
        _a16_body,
        grid=(8,),
        in_specs=[pl.BlockSpec((1280, D), lambda i: (i, 0)),
                  pl.BlockSpec((16, D), lambda i: (0, 0))],
        out_specs=pl.BlockSpec((1280, 16), lambda i: (i, 0)),
        out_shape=jax.ShapeDtypeStruct((N_PAD, 16), F32),
    )(e2p, w2p)

    m_part, s_part, att = _sc_stats(a16.reshape(N_PAD * 16), hidx, tidx)

    m2, inv2 = pl.pallas_call(
        _combine_body,
        grid=(8,),
        in_specs=[pl.BlockSpec((NW, 1280), lambda i: (0, i)),
                  pl.BlockSpec((NW, 1280), lambda i: (0, i))],
        out_specs=[pl.BlockSpec((1, 1280), lambda i: (0, i)),
                   pl.BlockSpec((1, 1280), lambda i: (0, i))],
        out_shape=[jax.ShapeDtypeStruct((1, N_PAD), F32),
                   jax.ShapeDtypeStruct((1, N_PAD), F32)],
    )(m_part, s_part)

    headc = jnp.stack([
        jnp.clip(head - p * NR_RANGE, 0, NR_RANGE - 1).reshape(NW, NCH, CH)
        for p in range(2)])
    parts = _sc_agg(tab, head.reshape(NW, EPT), headc, trow, att,
                    m2.reshape(N_PAD), inv2.reshape(N_PAD))

    entity_agg = pl.pallas_call(
        _final_body,
        grid=(8,),
        in_specs=[pl.BlockSpec((1280, D), lambda i: (i, 0)),
                  pl.BlockSpec((1280, D), lambda i: (i, 0))],
        out_specs=pl.BlockSpec((1280, D), lambda i: (i, 0)),
        out_shape=jax.ShapeDtypeStruct((N_PAD, D), F32),
    )(parts[0], parts[1])[:N_ENT]

    user_agg = pl.pallas_call(
        _user_body,
        grid=(16,),
        in_specs=[pl.BlockSpec((256, N_ENT), lambda i: (i, 0)),
                  pl.BlockSpec((N_ENT, D), lambda i: (0, 0)),
                  pl.BlockSpec((256, D), lambda i: (i, 0)),
                  pl.BlockSpec((16, D), lambda i: (0, 0))],
        out_specs=pl.BlockSpec((256, D), lambda i: (i, 0)),
        out_shape=jax.ShapeDtypeStruct((N_USERS, D), F32),
    )(interact_mat, entity_emb, user_emb, wp)

    return entity_agg, user_agg
